# baseline (speedup 1.0000x reference)
import functools

import jax
import jax.numpy as jnp
from jax import lax
from jax.experimental import pallas as pl
from jax.experimental.pallas import tpu as pltpu
from jax.experimental.pallas import tpu_sc as plsc

NC = 2
NS = 16
CH = 64
NBUF = 4
GB = 16


def _sc_aggregate(n_pad, d, n0):
    rows_per_tile = n_pad // NS
    ngroup = n0 // GB
    mesh = plsc.VectorSubcoreMesh(core_axis_name="c", subcore_axis_name="s")

    @functools.partial(
        pl.kernel,
        mesh=mesh,
        out_type=jax.ShapeDtypeStruct((n_pad, d), jnp.float32),
        scratch_types=[
            pltpu.VMEM((2 * GB, CH), jnp.int32),
            pltpu.VMEM((2 * GB, CH), jnp.int32),
            pltpu.VMEM((NBUF, CH, d), jnp.float32),
            pltpu.VMEM_SHARED((n_pad, d), jnp.float32),
            pltpu.SemaphoreType.DMA((NBUF,)),
            pltpu.SemaphoreType.DMA((NBUF,)),
            pltpu.SemaphoreType.DMA,
        ],
    )
    def body(src_hbm, dst_hbm, x_hbm, out_hbm, sidx, didx, rows, agg, gsem, ssem, isem):
        cid = lax.axis_index("c")
        sid = lax.axis_index("s")

        @pl.when(cid == 0)
        def _work():
            with jax.named_scope("zero_acc"):
                def zrow(r, _):
                    def zcol(c, _):
                        rows[0, r, pl.ds(c * 16, 16)] = jnp.zeros((16,), jnp.float32)
                        return 0
                    return lax.fori_loop(0, d // 16, zcol, 0)
                lax.fori_loop(0, CH, zrow, 0)

                def zcopy(t, _):
                    pltpu.sync_copy(rows.at[0], agg.at[pl.ds(sid * rows_per_tile + t * CH, CH)])
                    return 0
                lax.fori_loop(0, rows_per_tile // CH, zcopy, 0)

            start = sid * n0

            with jax.named_scope("idx_prologue"):
                pltpu.sync_copy(src_hbm.at[pl.ds(start, GB)], sidx.at[pl.ds(0, GB)])
                pltpu.sync_copy(dst_hbm.at[pl.ds(start, GB)], didx.at[pl.ds(0, GB)])
                plsc.subcore_barrier()
                pltpu.async_copy(src_hbm.at[pl.ds(start + GB, GB)], sidx.at[pl.ds(GB, GB)], isem)
                pltpu.async_copy(dst_hbm.at[pl.ds(start + GB, GB)], didx.at[pl.ds(GB, GB)], isem)

            def swait(b):
                pltpu.make_async_copy(rows.at[b], agg.at[didx.at[0]], ssem.at[b]).wait()

            def iwait():
                pltpu.make_async_copy(src_hbm.at[pl.ds(0, GB)], sidx.at[pl.ds(0, GB)], isem).wait()
                pltpu.make_async_copy(dst_hbm.at[pl.ds(0, GB)], didx.at[pl.ds(0, GB)], isem).wait()

            with jax.named_scope("edge_loop"):
                def group_body(gi, _):
                    slot = lax.rem(gi, 2)
                    base = slot * GB

                    @pl.when(gi > 0)
                    def _():
                        iwait()

                    for p in range(NBUF - 1):
                        pltpu.async_copy(x_hbm.at[sidx.at[base + p]], rows.at[p], gsem.at[p])

                    def kk_body(kk, _):
                        for b in range(NBUF):
                            k = kk * NBUF + b
                            row = base + k
                            bn = (b + NBUF - 1) % NBUF
                            pltpu.make_async_copy(x_hbm.at[sidx.at[row]], rows.at[b], gsem.at[b]).wait()
                            pltpu.async_copy(rows.at[b], agg.at[didx.at[row]], ssem.at[b], add=True)

                            @pl.when(k > 0)
                            def _():
                                swait(bn)

                            @pl.when(k < GB - NBUF + 1)
                            def _():
                                pltpu.async_copy(x_hbm.at[sidx.at[row + NBUF - 1]], rows.at[bn], gsem.at[bn])
                        return 0
                    lax.fori_loop(0, GB // NBUF, kk_body, 0)
                    swait((GB - 1) % NBUF)

                    @pl.when(gi + 2 < ngroup)
                    def _():
                        off = start + (gi + 2) * GB
                        pltpu.async_copy(src_hbm.at[pl.ds(off, GB)], sidx.at[pl.ds(base, GB)], isem)
                        pltpu.async_copy(dst_hbm.at[pl.ds(off, GB)], didx.at[pl.ds(base, GB)], isem)
                    return 0
                lax.fori_loop(0, ngroup, group_body, 0)
                plsc.subcore_barrier()

            with jax.named_scope("writeout"):
                def wcopy(t, _):
                    r0 = sid * rows_per_tile + t * CH
                    pltpu.sync_copy(agg.at[pl.ds(r0, CH)], rows.at[0])
                    pltpu.sync_copy(rows.at[0], out_hbm.at[pl.ds(r0, CH)])
                    return 0
                lax.fori_loop(0, rows_per_tile // CH, wcopy, 0)

    return body


def _mlp(x, a0, W1, b1, W2, b2, eps, blk):
    n, d = x.shape

    def body(eps_ref, x_ref, a0_ref, w1_ref, b1_ref, w2_ref, b2_ref, o_ref):
        s = 1.0 + eps_ref[0, 0]
        out = s * x_ref[...] + a0_ref[...]
        h = jnp.dot(out, w1_ref[...], preferred_element_type=jnp.float32)
        h = jnp.maximum(h + b1_ref[...], 0.0)
        o_ref[...] = jnp.dot(h, w2_ref[...], preferred_element_type=jnp.float32) + b2_ref[...]

    return pl.pallas_call(
        body,
        grid=(n // blk,),
        in_specs=[
            pl.BlockSpec(memory_space=pltpu.SMEM),
            pl.BlockSpec((blk, d), lambda i: (i, 0)),
            pl.BlockSpec((blk, d), lambda i: (i, 0)),
            pl.BlockSpec((d, d), lambda i: (0, 0)),
            pl.BlockSpec((1, d), lambda i: (0, 0)),
            pl.BlockSpec((d, d), lambda i: (0, 0)),
            pl.BlockSpec((1, d), lambda i: (0, 0)),
        ],
        out_specs=pl.BlockSpec((blk, d), lambda i: (i, 0)),
        out_shape=jax.ShapeDtypeStruct((n, d), jnp.float32),
    )(eps, x, a0, W1, b1.reshape(1, d), W2, b2.reshape(1, d))


def kernel(x, edge_index, W1, b1, W2, b2, eps):
    n, d = x.shape
    e = edge_index.shape[1]

    n0 = -(-e // (NS * CH * GB)) * GB
    e_pad = NS * n0 * CH
    n_pad = -(-(n + 1) // (NS * CH)) * (NS * CH)

    src = edge_index[0]
    dst = edge_index[1]
    pad = e_pad - e
    src_p = jnp.concatenate([src, jnp.zeros((pad,), jnp.int32)]).reshape(-1, CH)
    dst_p = jnp.concatenate([dst, jnp.full((pad,), n, jnp.int32)]).reshape(-1, CH)

    partial = _sc_aggregate(n_pad, d, n0)(src_p, dst_p, x)

    blk = 2000 if n % 2000 == 0 else (1250 if n % 1250 == 0 else n)
    return _mlp(x, partial[:n], W1, b1, W2, b2, eps, blk)


# speedup vs baseline: 1.4556x; 1.4556x over previous
import functools

import jax
import jax.numpy as jnp
from jax import lax
from jax.experimental import pallas as pl
from jax.experimental.pallas import tpu as pltpu
from jax.experimental.pallas import tpu_sc as plsc

NC = 2
NS = 16
NW = NC * NS
CH = 64
NBUF = 4
GB = 16
SC0_FRAC = 0.95


def _sc_aggregate(n_pad, d, n0, n1):
    rows_per_tile = n_pad // NS
    mesh = plsc.VectorSubcoreMesh(core_axis_name="c", subcore_axis_name="s")

    @functools.partial(
        pl.kernel,
        mesh=mesh,
        out_type=jax.ShapeDtypeStruct((NC, n_pad, d), jnp.float32),
        scratch_types=[
            pltpu.VMEM((2 * GB, CH), jnp.int32),
            pltpu.VMEM((2 * GB, CH), jnp.int32),
            pltpu.VMEM((NBUF, CH, d), jnp.float32),
            pltpu.VMEM_SHARED((n_pad, d), jnp.float32),
            pltpu.SemaphoreType.DMA((NBUF,)),
            pltpu.SemaphoreType.DMA((NBUF,)),
            pltpu.SemaphoreType.DMA,
        ],
    )
    def body(src_hbm, dst_hbm, x_hbm, out_hbm, sidx, didx, rows, agg, gsem, ssem, isem):
        cid = lax.axis_index("c")
        sid = lax.axis_index("s")

        with jax.named_scope("zero_acc"):
            def zrow(r, _):
                def zcol(c, _):
                    rows[0, r, pl.ds(c * 16, 16)] = jnp.zeros((16,), jnp.float32)
                    return 0
                return lax.fori_loop(0, d // 16, zcol, 0)
            lax.fori_loop(0, CH, zrow, 0)

            def zcopy(t, _):
                pltpu.sync_copy(rows.at[0], agg.at[pl.ds(sid * rows_per_tile + t * CH, CH)])
                return 0
            lax.fori_loop(0, rows_per_tile // CH, zcopy, 0)

        start = jnp.where(cid == 0, sid * n0, NS * n0 + sid * n1)
        ngroup_w = jnp.where(cid == 0, n0 // GB, n1 // GB)

        with jax.named_scope("idx_prologue"):
            pltpu.sync_copy(src_hbm.at[pl.ds(start, GB)], sidx.at[pl.ds(0, GB)])
            pltpu.sync_copy(dst_hbm.at[pl.ds(start, GB)], didx.at[pl.ds(0, GB)])
            plsc.subcore_barrier()

            @pl.when(ngroup_w > 1)
            def _():
                pltpu.async_copy(src_hbm.at[pl.ds(start + GB, GB)], sidx.at[pl.ds(GB, GB)], isem)
                pltpu.async_copy(dst_hbm.at[pl.ds(start + GB, GB)], didx.at[pl.ds(GB, GB)], isem)

        def swait(b):
            pltpu.make_async_copy(rows.at[b], agg.at[didx.at[0]], ssem.at[b]).wait()

        def iwait():
            pltpu.make_async_copy(src_hbm.at[pl.ds(0, GB)], sidx.at[pl.ds(0, GB)], isem).wait()
            pltpu.make_async_copy(dst_hbm.at[pl.ds(0, GB)], didx.at[pl.ds(0, GB)], isem).wait()

        with jax.named_scope("edge_loop"):
            def group_body(gi, _):
                slot = lax.rem(gi, 2)
                base = slot * GB

                @pl.when(gi > 0)
                def _():
                    iwait()

                for p in range(NBUF - 1):
                    pltpu.async_copy(x_hbm.at[sidx.at[base + p]], rows.at[p], gsem.at[p])

                def kk_body(kk, _):
                    for b in range(NBUF):
                        k = kk * NBUF + b
                        row = base + k
                        bn = (b + NBUF - 1) % NBUF
                        pltpu.make_async_copy(x_hbm.at[sidx.at[row]], rows.at[b], gsem.at[b]).wait()
                        pltpu.async_copy(rows.at[b], agg.at[didx.at[row]], ssem.at[b], add=True)

                        @pl.when(k > 0)
                        def _():
                            swait(bn)

                        @pl.when(k < GB - NBUF + 1)
                        def _():
                            pltpu.async_copy(x_hbm.at[sidx.at[row + NBUF - 1]], rows.at[bn], gsem.at[bn])
                    return 0
                lax.fori_loop(0, GB // NBUF, kk_body, 0)
                swait((GB - 1) % NBUF)

                @pl.when(gi + 2 < ngroup_w)
                def _():
                    off = start + (gi + 2) * GB
                    pltpu.async_copy(src_hbm.at[pl.ds(off, GB)], sidx.at[pl.ds(base, GB)], isem)
                    pltpu.async_copy(dst_hbm.at[pl.ds(off, GB)], didx.at[pl.ds(base, GB)], isem)
                return 0
            lax.fori_loop(0, ngroup_w, group_body, 0)
            plsc.subcore_barrier()

        with jax.named_scope("writeout"):
            def wcopy(t, _):
                r0 = sid * rows_per_tile + t * CH
                pltpu.sync_copy(agg.at[pl.ds(r0, CH)], rows.at[0])
                pltpu.sync_copy(rows.at[0], out_hbm.at[cid, pl.ds(r0, CH)])
                return 0
            lax.fori_loop(0, rows_per_tile // CH, wcopy, 0)

    return body


def _mlp(x, a0, a1, W1, b1, W2, b2, eps, blk):
    n, d = x.shape

    def body(eps_ref, x_ref, a0_ref, a1_ref, w1_ref, b1_ref, w2_ref, b2_ref, o_ref):
        s = 1.0 + eps_ref[0, 0]
        out = s * x_ref[...] + a0_ref[...] + a1_ref[...]
        h = jnp.dot(out, w1_ref[...], preferred_element_type=jnp.float32)
        h = jnp.maximum(h + b1_ref[...], 0.0)
        o_ref[...] = jnp.dot(h, w2_ref[...], preferred_element_type=jnp.float32) + b2_ref[...]

    return pl.pallas_call(
        body,
        grid=(n // blk,),
        in_specs=[
            pl.BlockSpec(memory_space=pltpu.SMEM),
            pl.BlockSpec((blk, d), lambda i: (i, 0)),
            pl.BlockSpec((blk, d), lambda i: (i, 0)),
            pl.BlockSpec((blk, d), lambda i: (i, 0)),
            pl.BlockSpec((d, d), lambda i: (0, 0)),
            pl.BlockSpec((1, d), lambda i: (0, 0)),
            pl.BlockSpec((d, d), lambda i: (0, 0)),
            pl.BlockSpec((1, d), lambda i: (0, 0)),
        ],
        out_specs=pl.BlockSpec((blk, d), lambda i: (i, 0)),
        out_shape=jax.ShapeDtypeStruct((n, d), jnp.float32),
    )(eps, x, a0, a1, W1, b1.reshape(1, d), W2, b2.reshape(1, d))


def kernel(x, edge_index, W1, b1, W2, b2, eps):
    n, d = x.shape
    e = edge_index.shape[1]

    pair = 2 * (-(-e // (NW * CH * GB)) * GB)
    n0 = GB * max(1, min(pair // GB - 1, round(SC0_FRAC * pair / GB)))
    n1 = pair - n0
    e_pad = NS * pair * CH
    n_pad = -(-(n + 1) // (NS * CH)) * (NS * CH)

    src = edge_index[0]
    dst = edge_index[1]
    pad = e_pad - e
    src_p = jnp.concatenate([src, jnp.zeros((pad,), jnp.int32)]).reshape(-1, CH)
    dst_p = jnp.concatenate([dst, jnp.full((pad,), n, jnp.int32)]).reshape(-1, CH)

    partial = _sc_aggregate(n_pad, d, n0, n1)(src_p, dst_p, x)

    blk = 2000 if n % 2000 == 0 else (1250 if n % 1250 == 0 else n)
    return _mlp(x, partial[0, :n], partial[1, :n], W1, b1, W2, b2, eps, blk)


# speedup vs baseline: 2.4476x; 1.6815x over previous
import functools

import jax
import jax.numpy as jnp
from jax import lax
from jax.experimental import pallas as pl
from jax.experimental.pallas import tpu as pltpu
from jax.experimental.pallas import tpu_sc as plsc

NC = 2
NS = 16
NW = NC * NS
CH = 64
NBUF = 4
GB = 16
SC0_FRAC = 0.95


def _sc_aggregate(n_pad, d, n0, n1):
    rows_per_tile = n_pad // NS
    mesh = plsc.VectorSubcoreMesh(core_axis_name="c", subcore_axis_name="s")

    @functools.partial(
        pl.kernel,
        mesh=mesh,
        out_type=jax.ShapeDtypeStruct((NC, n_pad, d), jnp.float32),
        scratch_types=[
            pltpu.VMEM((2 * GB, CH), jnp.int32),
            pltpu.VMEM((2 * GB, CH), jnp.int32),
            pltpu.VMEM((NBUF, CH, d), jnp.float32),
            pltpu.VMEM_SHARED((n_pad, d), jnp.float32),
            pltpu.SemaphoreType.DMA((NBUF,)),
            pltpu.SemaphoreType.DMA((NBUF,)),
            pltpu.SemaphoreType.DMA,
        ],
    )
    def body(src_hbm, dst_hbm, x_hbm, out_hbm, sidx, didx, rows, agg, gsem, ssem, isem):
        cid = lax.axis_index("c")
        sid = lax.axis_index("s")

        with jax.named_scope("zero_acc"):
            def zrow(r, _):
                def zcol(c, _):
                    rows[0, r, pl.ds(c * 16, 16)] = jnp.zeros((16,), jnp.float32)
                    return 0
                return lax.fori_loop(0, d // 16, zcol, 0)
            lax.fori_loop(0, CH, zrow, 0)

            def zcopy(t, _):
                pltpu.sync_copy(rows.at[0], agg.at[pl.ds(sid * rows_per_tile + t * CH, CH)])
                return 0
            lax.fori_loop(0, rows_per_tile // CH, zcopy, 0)

        start = jnp.where(cid == 0, sid * n0, NS * n0 + sid * n1)
        ngroup_w = jnp.where(cid == 0, n0 // GB, n1 // GB)

        with jax.named_scope("idx_prologue"):
            pltpu.sync_copy(src_hbm.at[pl.ds(start, GB)], sidx.at[pl.ds(0, GB)])
            pltpu.sync_copy(dst_hbm.at[pl.ds(start, GB)], didx.at[pl.ds(0, GB)])
            plsc.subcore_barrier()

            @pl.when(ngroup_w > 1)
            def _():
                pltpu.async_copy(src_hbm.at[pl.ds(start + GB, GB)], sidx.at[pl.ds(GB, GB)], isem)
                pltpu.async_copy(dst_hbm.at[pl.ds(start + GB, GB)], didx.at[pl.ds(GB, GB)], isem)

        def swait(b):
            pltpu.make_async_copy(rows.at[b], agg.at[didx.at[0]], ssem.at[b]).wait()

        def iwait():
            pltpu.make_async_copy(src_hbm.at[pl.ds(0, GB)], sidx.at[pl.ds(0, GB)], isem).wait()
            pltpu.make_async_copy(dst_hbm.at[pl.ds(0, GB)], didx.at[pl.ds(0, GB)], isem).wait()

        with jax.named_scope("edge_loop"):
            def group_body(gi, _):
                slot = lax.rem(gi, 2)
                base = slot * GB

                @pl.when(gi > 0)
                def _():
                    iwait()

                for p in range(NBUF - 1):
                    pltpu.async_copy(x_hbm.at[sidx.at[base + p]], rows.at[p], gsem.at[p])

                def kk_body(kk, _):
                    for b in range(NBUF):
                        k = kk * NBUF + b
                        row = base + k
                        bn = (b + NBUF - 1) % NBUF
                        pltpu.make_async_copy(x_hbm.at[sidx.at[row]], rows.at[b], gsem.at[b]).wait()
                        pltpu.async_copy(rows.at[b], agg.at[didx.at[row]], ssem.at[b], add=True)

                        @pl.when(k > 0)
                        def _():
                            swait(bn)

                        @pl.when(k < GB - NBUF + 1)
                        def _():
                            pltpu.async_copy(x_hbm.at[sidx.at[row + NBUF - 1]], rows.at[bn], gsem.at[bn])
                    return 0
                lax.fori_loop(0, GB // NBUF, kk_body, 0)
                swait((GB - 1) % NBUF)

                @pl.when(gi + 2 < ngroup_w)
                def _():
                    off = start + (gi + 2) * GB
                    pltpu.async_copy(src_hbm.at[pl.ds(off, GB)], sidx.at[pl.ds(base, GB)], isem)
                    pltpu.async_copy(dst_hbm.at[pl.ds(off, GB)], didx.at[pl.ds(base, GB)], isem)
                return 0
            lax.fori_loop(0, ngroup_w, group_body, 0)
            plsc.subcore_barrier()

        with jax.named_scope("writeout"):
            def wcopy(t, _):
                r0 = sid * rows_per_tile + t * CH
                pltpu.sync_copy(agg.at[pl.ds(r0, CH)], rows.at[0])
                pltpu.sync_copy(rows.at[0], out_hbm.at[cid, pl.ds(r0, CH)])
                return 0
            lax.fori_loop(0, rows_per_tile // CH, wcopy, 0)

    return body


def _mlp(x, a0, a1, W1, b1, W2, b2, eps, blk):
    n, d = x.shape

    def body(eps_ref, x_ref, a0_ref, a1_ref, w1_ref, b1_ref, w2_ref, b2_ref, o_ref):
        s = 1.0 + eps_ref[0, 0]
        out = s * x_ref[...] + a0_ref[...] + a1_ref[...]
        h = jnp.dot(out, w1_ref[...], preferred_element_type=jnp.float32)
        h = jnp.maximum(h + b1_ref[...], 0.0)
        o_ref[...] = jnp.dot(h, w2_ref[...], preferred_element_type=jnp.float32) + b2_ref[...]

    return pl.pallas_call(
        body,
        grid=(n // blk,),
        in_specs=[
            pl.BlockSpec(memory_space=pltpu.SMEM),
            pl.BlockSpec((blk, d), lambda i: (i, 0)),
            pl.BlockSpec((blk, d), lambda i: (i, 0)),
            pl.BlockSpec((blk, d), lambda i: (i, 0)),
            pl.BlockSpec((d, d), lambda i: (0, 0)),
            pl.BlockSpec((1, d), lambda i: (0, 0)),
            pl.BlockSpec((d, d), lambda i: (0, 0)),
            pl.BlockSpec((1, d), lambda i: (0, 0)),
        ],
        out_specs=pl.BlockSpec((blk, d), lambda i: (i, 0)),
        out_shape=jax.ShapeDtypeStruct((n, d), jnp.float32),
    )(eps, x, a0, a1, W1, b1.reshape(1, d), W2, b2.reshape(1, d))


def kernel(x, edge_index, W1, b1, W2, b2, eps):
    n, d = x.shape
    e = edge_index.shape[1]

    pair = 2 * (-(-e // (NW * CH * GB)) * GB)
    n0 = GB * max(1, min(pair // GB - 1, round(SC0_FRAC * pair / GB)))
    n1 = pair - n0
    e_pad = NS * pair * CH
    n_pad = -(-(n + 1) // (NS * CH)) * (NS * CH)

    src = edge_index[0]
    dst = edge_index[1]
    pad = e_pad - e
    pad_src = jnp.arange(pad, dtype=jnp.int32) % n
    pad_dst = n + jnp.arange(pad, dtype=jnp.int32) % (n_pad - n)
    src_p = jnp.concatenate([src, pad_src]).reshape(-1, CH)
    dst_p = jnp.concatenate([dst, pad_dst]).reshape(-1, CH)

    partial = _sc_aggregate(n_pad, d, n0, n1)(src_p, dst_p, x)

    blk = 2000 if n % 2000 == 0 else (1250 if n % 1250 == 0 else n)
    return _mlp(x, partial[0, :n], partial[1, :n], W1, b1, W2, b2, eps, blk)


# speedup vs baseline: 3.7352x; 1.5260x over previous
import functools

import jax
import jax.numpy as jnp
from jax import lax
from jax.experimental import pallas as pl
from jax.experimental.pallas import tpu as pltpu
from jax.experimental.pallas import tpu_sc as plsc

NC = 2
NS = 16
NW = NC * NS
CH = 64
NBUF = 4
GB = 16
SC0_FRAC = 0.5


def _sc_aggregate(n_pad, d, n0, n1):
    rows_per_tile = n_pad // NS
    mesh = plsc.VectorSubcoreMesh(core_axis_name="c", subcore_axis_name="s")

    @functools.partial(
        pl.kernel,
        mesh=mesh,
        out_type=jax.ShapeDtypeStruct((NC, n_pad, d), jnp.float32),
        scratch_types=[
            pltpu.VMEM((2 * GB, CH), jnp.int32),
            pltpu.VMEM((2 * GB, CH), jnp.int32),
            pltpu.VMEM((NBUF, CH, d), jnp.float32),
            pltpu.VMEM_SHARED((n_pad, d), jnp.float32),
            pltpu.SemaphoreType.DMA((NBUF,)),
            pltpu.SemaphoreType.DMA((NBUF,)),
            pltpu.SemaphoreType.DMA,
        ],
    )
    def body(src_hbm, dst_hbm, x_hbm, out_hbm, sidx, didx, rows, agg, gsem, ssem, isem):
        cid = lax.axis_index("c")
        sid = lax.axis_index("s")

        with jax.named_scope("zero_acc"):
            def zrow(r, _):
                def zcol(c, _):
                    rows[0, r, pl.ds(c * 16, 16)] = jnp.zeros((16,), jnp.float32)
                    return 0
                return lax.fori_loop(0, d // 16, zcol, 0)
            lax.fori_loop(0, CH, zrow, 0)

            def zcopy(t, _):
                pltpu.sync_copy(rows.at[0], agg.at[pl.ds(sid * rows_per_tile + t * CH, CH)])
                return 0
            lax.fori_loop(0, rows_per_tile // CH, zcopy, 0)

        start = jnp.where(cid == 0, sid * n0, NS * n0 + sid * n1)
        ngroup_w = jnp.where(cid == 0, n0 // GB, n1 // GB)

        with jax.named_scope("idx_prologue"):
            pltpu.sync_copy(src_hbm.at[pl.ds(start, GB)], sidx.at[pl.ds(0, GB)])
            pltpu.sync_copy(dst_hbm.at[pl.ds(start, GB)], didx.at[pl.ds(0, GB)])
            plsc.subcore_barrier()

            @pl.when(ngroup_w > 1)
            def _():
                pltpu.async_copy(src_hbm.at[pl.ds(start + GB, GB)], sidx.at[pl.ds(GB, GB)], isem)
                pltpu.async_copy(dst_hbm.at[pl.ds(start + GB, GB)], didx.at[pl.ds(GB, GB)], isem)

        def swait(b):
            pltpu.make_async_copy(rows.at[b], agg.at[didx.at[0]], ssem.at[b]).wait()

        def iwait():
            pltpu.make_async_copy(src_hbm.at[pl.ds(0, GB)], sidx.at[pl.ds(0, GB)], isem).wait()
            pltpu.make_async_copy(dst_hbm.at[pl.ds(0, GB)], didx.at[pl.ds(0, GB)], isem).wait()

        with jax.named_scope("edge_loop"):
            def group_body(gi, _):
                slot = lax.rem(gi, 2)
                base = slot * GB

                @pl.when(gi > 0)
                def _():
                    iwait()

                for p in range(NBUF - 1):
                    pltpu.async_copy(x_hbm.at[sidx.at[base + p]], rows.at[p], gsem.at[p])

                def kk_body(kk, _):
                    for b in range(NBUF):
                        k = kk * NBUF + b
                        row = base + k
                        bn = (b + NBUF - 1) % NBUF
                        pltpu.make_async_copy(x_hbm.at[sidx.at[row]], rows.at[b], gsem.at[b]).wait()
                        pltpu.async_copy(rows.at[b], agg.at[didx.at[row]], ssem.at[b], add=True)

                        @pl.when(k > 0)
                        def _():
                            swait(bn)

                        @pl.when(k < GB - NBUF + 1)
                        def _():
                            pltpu.async_copy(x_hbm.at[sidx.at[row + NBUF - 1]], rows.at[bn], gsem.at[bn])
                    return 0
                lax.fori_loop(0, GB // NBUF, kk_body, 0)
                swait((GB - 1) % NBUF)

                @pl.when(gi + 2 < ngroup_w)
                def _():
                    off = start + (gi + 2) * GB
                    pltpu.async_copy(src_hbm.at[pl.ds(off, GB)], sidx.at[pl.ds(base, GB)], isem)
                    pltpu.async_copy(dst_hbm.at[pl.ds(off, GB)], didx.at[pl.ds(base, GB)], isem)
                return 0
            lax.fori_loop(0, ngroup_w, group_body, 0)
            plsc.subcore_barrier()

        with jax.named_scope("writeout"):
            def wcopy(t, _):
                r0 = sid * rows_per_tile + t * CH
                pltpu.sync_copy(agg.at[pl.ds(r0, CH)], rows.at[0])
                pltpu.sync_copy(rows.at[0], out_hbm.at[cid, pl.ds(r0, CH)])
                return 0
            lax.fori_loop(0, rows_per_tile // CH, wcopy, 0)

    return body


def _mlp(x, a0, a1, W1, b1, W2, b2, eps, blk):
    n, d = x.shape

    def body(eps_ref, x_ref, a0_ref, a1_ref, w1_ref, b1_ref, w2_ref, b2_ref, o_ref):
        s = 1.0 + eps_ref[0, 0]
        out = s * x_ref[...] + a0_ref[...] + a1_ref[...]
        h = jnp.dot(out, w1_ref[...], preferred_element_type=jnp.float32)
        h = jnp.maximum(h + b1_ref[...], 0.0)
        o_ref[...] = jnp.dot(h, w2_ref[...], preferred_element_type=jnp.float32) + b2_ref[...]

    return pl.pallas_call(
        body,
        grid=(n // blk,),
        in_specs=[
            pl.BlockSpec(memory_space=pltpu.SMEM),
            pl.BlockSpec((blk, d), lambda i: (i, 0)),
            pl.BlockSpec((blk, d), lambda i: (i, 0)),
            pl.BlockSpec((blk, d), lambda i: (i, 0)),
            pl.BlockSpec((d, d), lambda i: (0, 0)),
            pl.BlockSpec((1, d), lambda i: (0, 0)),
            pl.BlockSpec((d, d), lambda i: (0, 0)),
            pl.BlockSpec((1, d), lambda i: (0, 0)),
        ],
        out_specs=pl.BlockSpec((blk, d), lambda i: (i, 0)),
        out_shape=jax.ShapeDtypeStruct((n, d), jnp.float32),
    )(eps, x, a0, a1, W1, b1.reshape(1, d), W2, b2.reshape(1, d))


def kernel(x, edge_index, W1, b1, W2, b2, eps):
    n, d = x.shape
    e = edge_index.shape[1]

    pair = 2 * (-(-e // (NW * CH * GB)) * GB)
    n0 = GB * max(1, min(pair // GB - 1, round(SC0_FRAC * pair / GB)))
    n1 = pair - n0
    e_pad = NS * pair * CH
    n_pad = -(-(n + 1) // (NS * CH)) * (NS * CH)

    src = edge_index[0]
    dst = edge_index[1]
    pad = e_pad - e
    pad_src = jnp.arange(pad, dtype=jnp.int32) % n
    pad_dst = n + jnp.arange(pad, dtype=jnp.int32) % (n_pad - n)
    src_p = jnp.concatenate([src, pad_src]).reshape(-1, CH)
    dst_p = jnp.concatenate([dst, pad_dst]).reshape(-1, CH)

    partial = _sc_aggregate(n_pad, d, n0, n1)(src_p, dst_p, x)

    blk = 2000 if n % 2000 == 0 else (1250 if n % 1250 == 0 else n)
    return _mlp(x, partial[0, :n], partial[1, :n], W1, b1, W2, b2, eps, blk)


# speedup vs baseline: 4.0511x; 1.0846x over previous
import functools

import jax
import jax.numpy as jnp
from jax import lax
from jax.experimental import pallas as pl
from jax.experimental.pallas import tpu as pltpu
from jax.experimental.pallas import tpu_sc as plsc

NC = 2
NS = 16
NW = NC * NS
CH = 64
NBUF = 4
GB = 16
SC0_FRAC = 0.5


def _sc_aggregate(n_pad, d, n0, n1):
    rows_per_tile = n_pad // NS
    mesh = plsc.VectorSubcoreMesh(core_axis_name="c", subcore_axis_name="s")

    @functools.partial(
        pl.kernel,
        mesh=mesh,
        out_type=jax.ShapeDtypeStruct((NC, n_pad, d), jnp.float32),
        scratch_types=[
            pltpu.VMEM((2 * GB, CH), jnp.int32),
            pltpu.VMEM((2 * GB, CH), jnp.int32),
            pltpu.VMEM((NBUF, CH, d), jnp.float32),
            pltpu.VMEM_SHARED((n_pad, d), jnp.float32),
            pltpu.SemaphoreType.DMA((NBUF,)),
            pltpu.SemaphoreType.DMA((NBUF,)),
            pltpu.SemaphoreType.DMA,
        ],
    )
    def body(ei_hbm, x_hbm, out_hbm, sidx, didx, rows, agg, gsem, ssem, isem):
        cid = lax.axis_index("c")
        sid = lax.axis_index("s")

        with jax.named_scope("zero_acc"):
            def zrow(r, _):
                def zcol(c, _):
                    rows[0, r, pl.ds(c * 16, 16)] = jnp.zeros((16,), jnp.float32)
                    return 0
                return lax.fori_loop(0, d // 16, zcol, 0)
            lax.fori_loop(0, CH, zrow, 0)

            def zcopy(t, _):
                pltpu.sync_copy(rows.at[0], agg.at[pl.ds(sid * rows_per_tile + t * CH, CH)])
                return 0
            lax.fori_loop(0, rows_per_tile // CH, zcopy, 0)

        start = jnp.where(cid == 0, sid * n0, NS * n0 + sid * n1)
        ngroup_w = jnp.where(cid == 0, n0 // GB, n1 // GB)

        with jax.named_scope("idx_prologue"):
            pltpu.sync_copy(ei_hbm.at[0, pl.ds(start, GB)], sidx.at[pl.ds(0, GB)])
            pltpu.sync_copy(ei_hbm.at[1, pl.ds(start, GB)], didx.at[pl.ds(0, GB)])
            plsc.subcore_barrier()

            @pl.when(ngroup_w > 1)
            def _():
                pltpu.async_copy(ei_hbm.at[0, pl.ds(start + GB, GB)], sidx.at[pl.ds(GB, GB)], isem)
                pltpu.async_copy(ei_hbm.at[1, pl.ds(start + GB, GB)], didx.at[pl.ds(GB, GB)], isem)

        def swait(b):
            pltpu.make_async_copy(rows.at[b], agg.at[didx.at[0]], ssem.at[b]).wait()

        def iwait():
            pltpu.make_async_copy(ei_hbm.at[0, pl.ds(0, GB)], sidx.at[pl.ds(0, GB)], isem).wait()
            pltpu.make_async_copy(ei_hbm.at[1, pl.ds(0, GB)], didx.at[pl.ds(0, GB)], isem).wait()

        with jax.named_scope("edge_loop"):
            def group_body(gi, _):
                slot = lax.rem(gi, 2)
                base = slot * GB

                @pl.when(gi > 0)
                def _():
                    iwait()

                for p in range(NBUF - 1):
                    pltpu.async_copy(x_hbm.at[sidx.at[base + p]], rows.at[p], gsem.at[p])

                def kk_body(kk, _):
                    for b in range(NBUF):
                        k = kk * NBUF + b
                        row = base + k
                        bn = (b + NBUF - 1) % NBUF
                        pltpu.make_async_copy(x_hbm.at[sidx.at[row]], rows.at[b], gsem.at[b]).wait()
                        pltpu.async_copy(rows.at[b], agg.at[didx.at[row]], ssem.at[b], add=True)

                        @pl.when(k > 0)
                        def _():
                            swait(bn)

                        @pl.when(k < GB - NBUF + 1)
                        def _():
                            pltpu.async_copy(x_hbm.at[sidx.at[row + NBUF - 1]], rows.at[bn], gsem.at[bn])
                    return 0
                lax.fori_loop(0, GB // NBUF, kk_body, 0)
                swait((GB - 1) % NBUF)

                @pl.when(gi + 2 < ngroup_w)
                def _():
                    off = start + (gi + 2) * GB
                    pltpu.async_copy(ei_hbm.at[0, pl.ds(off, GB)], sidx.at[pl.ds(base, GB)], isem)
                    pltpu.async_copy(ei_hbm.at[1, pl.ds(off, GB)], didx.at[pl.ds(base, GB)], isem)
                return 0
            lax.fori_loop(0, ngroup_w, group_body, 0)
            plsc.subcore_barrier()

        with jax.named_scope("writeout"):
            def wcopy(t, _):
                r0 = sid * rows_per_tile + t * CH
                pltpu.sync_copy(agg.at[pl.ds(r0, CH)], rows.at[0])
                pltpu.sync_copy(rows.at[0], out_hbm.at[cid, pl.ds(r0, CH)])
                return 0
            lax.fori_loop(0, rows_per_tile // CH, wcopy, 0)

    return body


def _mlp(x, partial, W1, b1, W2, b2, eps, blk):
    n, d = x.shape

    def body(eps_ref, x_ref, a0_ref, a1_ref, w1_ref, b1_ref, w2_ref, b2_ref, o_ref):
        s = 1.0 + eps_ref[0, 0]
        out = s * x_ref[...] + a0_ref[0] + a1_ref[0]
        h = jnp.dot(out, w1_ref[...], preferred_element_type=jnp.float32)
        h = jnp.maximum(h + b1_ref[...], 0.0)
        o_ref[...] = jnp.dot(h, w2_ref[...], preferred_element_type=jnp.float32) + b2_ref[...]

    return pl.pallas_call(
        body,
        grid=(n // blk,),
        in_specs=[
            pl.BlockSpec(memory_space=pltpu.SMEM),
            pl.BlockSpec((blk, d), lambda i: (i, 0)),
            pl.BlockSpec((1, blk, d), lambda i: (0, i, 0)),
            pl.BlockSpec((1, blk, d), lambda i: (1, i, 0)),
            pl.BlockSpec((d, d), lambda i: (0, 0)),
            pl.BlockSpec((1, d), lambda i: (0, 0)),
            pl.BlockSpec((d, d), lambda i: (0, 0)),
            pl.BlockSpec((1, d), lambda i: (0, 0)),
        ],
        out_specs=pl.BlockSpec((blk, d), lambda i: (i, 0)),
        out_shape=jax.ShapeDtypeStruct((n, d), jnp.float32),
    )(eps, x, partial, partial, W1, b1.reshape(1, d), W2, b2.reshape(1, d))


def kernel(x, edge_index, W1, b1, W2, b2, eps):
    n, d = x.shape
    e = edge_index.shape[1]

    pair = 2 * (-(-e // (NW * CH * GB)) * GB)
    n0 = GB * max(1, min(pair // GB - 1, round(SC0_FRAC * pair / GB)))
    n1 = pair - n0
    e_pad = NS * pair * CH
    n_pad = -(-(n + 1) // (NS * CH)) * (NS * CH)

    pad = e_pad - e
    pad_src = jnp.arange(pad, dtype=jnp.int32) % n
    pad_dst = n + jnp.arange(pad, dtype=jnp.int32) % (n_pad - n)
    ei_pad = jnp.concatenate(
        [edge_index, jnp.stack([pad_src, pad_dst])], axis=1
    ).reshape(2, -1, CH)

    partial = _sc_aggregate(n_pad, d, n0, n1)(ei_pad, x)

    blk = 2000 if n % 2000 == 0 else (1250 if n % 1250 == 0 else n)
    return _mlp(x, partial, W1, b1, W2, b2, eps, blk)


# speedup vs baseline: 4.1595x; 1.0267x over previous
import functools

import jax
import jax.numpy as jnp
from jax import lax
from jax.experimental import pallas as pl
from jax.experimental.pallas import tpu as pltpu
from jax.experimental.pallas import tpu_sc as plsc

NC = 2
NS = 16
NW = NC * NS
CH = 80
NBUF = 4
GB = 16
SC0_FRAC = 0.5


def _sc_aggregate(n_pad, d, n0, n1):
    rows_per_tile = n_pad // NS
    mesh = plsc.VectorSubcoreMesh(core_axis_name="c", subcore_axis_name="s")

    @functools.partial(
        pl.kernel,
        mesh=mesh,
        out_type=jax.ShapeDtypeStruct((NC, n_pad, d), jnp.float32),
        scratch_types=[
            pltpu.VMEM((2 * GB, CH), jnp.int32),
            pltpu.VMEM((2 * GB, CH), jnp.int32),
            pltpu.VMEM((NBUF, CH, d), jnp.float32),
            pltpu.VMEM_SHARED((n_pad, d), jnp.float32),
            pltpu.SemaphoreType.DMA((NBUF,)),
            pltpu.SemaphoreType.DMA((NBUF,)),
            pltpu.SemaphoreType.DMA,
        ],
    )
    def body(ei_hbm, x_hbm, out_hbm, sidx, didx, rows, agg, gsem, ssem, isem):
        cid = lax.axis_index("c")
        sid = lax.axis_index("s")

        with jax.named_scope("zero_acc"):
            def zrow(r, _):
                def zcol(c, _):
                    rows[0, r, pl.ds(c * 16, 16)] = jnp.zeros((16,), jnp.float32)
                    return 0
                return lax.fori_loop(0, d // 16, zcol, 0)
            lax.fori_loop(0, CH, zrow, 0)

            def zcopy(t, _):
                pltpu.sync_copy(rows.at[0], agg.at[pl.ds(sid * rows_per_tile + t * CH, CH)])
                return 0
            lax.fori_loop(0, rows_per_tile // CH, zcopy, 0)

        start = jnp.where(cid == 0, sid * n0, NS * n0 + sid * n1)
        ngroup_w = jnp.where(cid == 0, n0 // GB, n1 // GB)

        with jax.named_scope("idx_prologue"):
            pltpu.sync_copy(ei_hbm.at[0, pl.ds(start, GB)], sidx.at[pl.ds(0, GB)])
            pltpu.sync_copy(ei_hbm.at[1, pl.ds(start, GB)], didx.at[pl.ds(0, GB)])
            plsc.subcore_barrier()

            @pl.when(ngroup_w > 1)
            def _():
                pltpu.async_copy(ei_hbm.at[0, pl.ds(start + GB, GB)], sidx.at[pl.ds(GB, GB)], isem)
                pltpu.async_copy(ei_hbm.at[1, pl.ds(start + GB, GB)], didx.at[pl.ds(GB, GB)], isem)

        def swait(b):
            pltpu.make_async_copy(rows.at[b], agg.at[didx.at[0]], ssem.at[b]).wait()

        def iwait():
            pltpu.make_async_copy(ei_hbm.at[0, pl.ds(0, GB)], sidx.at[pl.ds(0, GB)], isem).wait()
            pltpu.make_async_copy(ei_hbm.at[1, pl.ds(0, GB)], didx.at[pl.ds(0, GB)], isem).wait()

        with jax.named_scope("edge_loop"):
            def group_body(gi, _):
                slot = lax.rem(gi, 2)
                base = slot * GB

                @pl.when(gi > 0)
                def _():
                    iwait()

                for p in range(NBUF - 1):
                    pltpu.async_copy(x_hbm.at[sidx.at[base + p]], rows.at[p], gsem.at[p])

                def kk_body(kk, _):
                    for b in range(NBUF):
                        k = kk * NBUF + b
                        row = base + k
                        bn = (b + NBUF - 1) % NBUF
                        pltpu.make_async_copy(x_hbm.at[sidx.at[row]], rows.at[b], gsem.at[b]).wait()
                        pltpu.async_copy(rows.at[b], agg.at[didx.at[row]], ssem.at[b], add=True)

                        @pl.when(k > 0)
                        def _():
                            swait(bn)

                        @pl.when(k < GB - NBUF + 1)
                        def _():
                            pltpu.async_copy(x_hbm.at[sidx.at[row + NBUF - 1]], rows.at[bn], gsem.at[bn])
                    return 0
                lax.fori_loop(0, GB // NBUF, kk_body, 0)
                swait((GB - 1) % NBUF)

                @pl.when(gi + 2 < ngroup_w)
                def _():
                    off = start + (gi + 2) * GB
                    pltpu.async_copy(ei_hbm.at[0, pl.ds(off, GB)], sidx.at[pl.ds(base, GB)], isem)
                    pltpu.async_copy(ei_hbm.at[1, pl.ds(off, GB)], didx.at[pl.ds(base, GB)], isem)
                return 0
            lax.fori_loop(0, ngroup_w, group_body, 0)
            plsc.subcore_barrier()

        with jax.named_scope("writeout"):
            def wcopy(t, _):
                r0 = sid * rows_per_tile + t * CH
                pltpu.sync_copy(agg.at[pl.ds(r0, CH)], rows.at[0])
                pltpu.sync_copy(rows.at[0], out_hbm.at[cid, pl.ds(r0, CH)])
                return 0
            lax.fori_loop(0, rows_per_tile // CH, wcopy, 0)

    return body


def _mlp(x, partial, W1, b1, W2, b2, eps, blk):
    n, d = x.shape

    def body(eps_ref, x_ref, a0_ref, a1_ref, w1_ref, b1_ref, w2_ref, b2_ref, o_ref):
        s = 1.0 + eps_ref[0, 0]
        out = s * x_ref[...] + a0_ref[0] + a1_ref[0]
        h = jnp.dot(out, w1_ref[...], preferred_element_type=jnp.float32)
        h = jnp.maximum(h + b1_ref[...], 0.0)
        o_ref[...] = jnp.dot(h, w2_ref[...], preferred_element_type=jnp.float32) + b2_ref[...]

    return pl.pallas_call(
        body,
        grid=(n // blk,),
        in_specs=[
            pl.BlockSpec(memory_space=pltpu.SMEM),
            pl.BlockSpec((blk, d), lambda i: (i, 0)),
            pl.BlockSpec((1, blk, d), lambda i: (0, i, 0)),
            pl.BlockSpec((1, blk, d), lambda i: (1, i, 0)),
            pl.BlockSpec((d, d), lambda i: (0, 0)),
            pl.BlockSpec((1, d), lambda i: (0, 0)),
            pl.BlockSpec((d, d), lambda i: (0, 0)),
            pl.BlockSpec((1, d), lambda i: (0, 0)),
        ],
        out_specs=pl.BlockSpec((blk, d), lambda i: (i, 0)),
        out_shape=jax.ShapeDtypeStruct((n, d), jnp.float32),
    )(eps, x, partial, partial, W1, b1.reshape(1, d), W2, b2.reshape(1, d))


def kernel(x, edge_index, W1, b1, W2, b2, eps):
    n, d = x.shape
    e = edge_index.shape[1]

    pair = 2 * (-(-e // (NW * CH * GB)) * GB)
    n0 = GB * max(1, min(pair // GB - 1, round(SC0_FRAC * pair / GB)))
    n1 = pair - n0
    e_pad = NS * pair * CH
    n_pad = -(-(n + 1) // (NS * CH)) * (NS * CH)

    pad = e_pad - e
    pad_src = jnp.arange(pad, dtype=jnp.int32) % n
    pad_dst = n + jnp.arange(pad, dtype=jnp.int32) % (n_pad - n)
    ei_pad = jnp.concatenate(
        [edge_index, jnp.stack([pad_src, pad_dst])], axis=1
    ).reshape(2, -1, CH)

    partial = _sc_aggregate(n_pad, d, n0, n1)(ei_pad, x)

    blk = 2000 if n % 2000 == 0 else (1250 if n % 1250 == 0 else n)
    return _mlp(x, partial, W1, b1, W2, b2, eps, blk)
